# Initial kernel scaffold; baseline (speedup 1.0000x reference)
import jax
import jax.numpy as jnp
from jax.experimental import pallas as pl


def kernel(attn, sims):
    raise NotImplementedError("write your pallas kernel here")


# speedup vs baseline: 3.1223x; 3.1223x over previous
import functools

import jax
import jax.numpy as jnp
from jax import lax
from jax.experimental import pallas as pl
from jax.experimental.pallas import tpu as pltpu
from jax.experimental.pallas import tpu_sc as plsc

B, HD, H, W, K = 2, 4, 224, 224, 49
SH = SW = 14
NCH = SH * SW
KSIZE, R = 7, 3
KPAD = 64

ROWS_PW = 14
ROWS_HB = 7
CHUNK_C = 28
SLAB_R = ROWS_HB + 6
SLAB_C = 40


def _weight_sc_kernel(sims_hbm, w_hbm, slab, outv):
    nc = 2
    wid = lax.axis_index("s") * nc + lax.axis_index("c")
    b = wid >> 4
    band = wid & 15
    r0 = band * ROWS_PW

    iota = lax.iota(jnp.int32, 16)
    kvs = [jnp.minimum(c * 16 + iota, K - 1) for c in range(4)]
    khv = [(kv * 9363) >> 16 for kv in kvs]
    kwv = [kv - kh * KSIZE for kv, kh in zip(kvs, khv)]

    def chunk_body(ci, _):
        hb = ci >> 3
        x0 = (ci & 7) * CHUNK_C
        rh0 = r0 + hb * ROWS_HB
        y0c = jnp.clip(rh0 - 3, 0, H - SLAB_R)
        x0a = pl.multiple_of(jnp.clip((x0 - 3) & ~7, 0, W - SLAB_C), 8)

        def dma_row(yy, _):
            y = y0c + yy
            pltpu.sync_copy(
                sims_hbm.at[b, y, pl.ds(x0a, SLAB_C), :],
                slab.at[yy],
            )
            return _

        lax.fori_loop(0, SLAB_R, dma_row, None)

        def row_body(hh, _):
            h = rh0 + hh
            sih = h >> 4
            hclip = jnp.clip(h - 3, 0, H - KSIZE)
            i0s = [hclip + khv[c] - y0c for c in range(4)]

            def col_body(wwi, _):
                w = x0 + wwi
                sjw = w >> 4
                wclip = jnp.clip(w - 3, 0, W - KSIZE)
                qrowv = jnp.full((16,), h - y0c, jnp.int32)
                qcolv = jnp.full((16,), w - x0a, jnp.int32)
                i1s = [(wclip - x0a) + kwv[c] for c in range(4)]
                accs = [jnp.zeros((16,), jnp.float32) for _ in range(4)]
                for di in (-1, 0, 1):
                    for dj in (-1, 0, 1):
                        g = (sih + di) * 14 + sjw + dj
                        lqv = jnp.full(
                            (16,), jnp.clip(g, 0, NCH - 1), jnp.int32
                        )
                        pi_vec = plsc.load_gather(slab, [qrowv, qcolv, lqv])
                        rowok = jnp.logical_and(sih + di >= 0, sih + di < SH)
                        ok = jnp.logical_and(
                            rowok,
                            jnp.logical_and(sjw + dj >= 0, sjw + dj < SW),
                        )
                        pi_d = jnp.where(ok, pi_vec, 0.0)
                        for c in range(4):
                            vals = plsc.load_gather(slab, [i0s[c], i1s[c], lqv])
                            accs[c] = accs[c] + pi_d * vals
                for c in range(4):
                    outv[hh, wwi, pl.ds(c * 16, 16)] = accs[c]
                return _

            lax.fori_loop(0, CHUNK_C, col_body, None)
            return _

        lax.fori_loop(0, ROWS_HB, row_body, None)
        pltpu.sync_copy(
            outv, w_hbm.at[b, pl.ds(rh0, ROWS_HB), pl.ds(x0, CHUNK_C), :]
        )
        return _

    lax.fori_loop(0, 2 * (W // CHUNK_C), chunk_body, None)


def _fuse_tc_kernel(attn_ref, w_ref, out_ref):
    a = attn_ref[0, 0]
    m = jnp.max(a, axis=-1, keepdims=True)
    e = jnp.exp(a - m)
    wt = w_ref[0][:, :, :K]
    o = e * wt
    s = jnp.sum(o, axis=-1, keepdims=True)
    out_ref[0, 0] = o * (1.0 / (1e-15 + s))


def kernel(attn, sims):
    sims_r = sims.reshape(B, H, W, NCH)

    mesh = plsc.VectorSubcoreMesh(core_axis_name="c", subcore_axis_name="s")
    weight_call = functools.partial(
        pl.kernel,
        mesh=mesh,
        compiler_params=pltpu.CompilerParams(
            use_tc_tiling_on_sc=False, needs_layout_passes=False
        ),
        out_type=jax.ShapeDtypeStruct((B, H, W, KPAD), jnp.float32),
        scratch_types=[
            pltpu.VMEM((SLAB_R, SLAB_C, NCH), jnp.float32),
            pltpu.VMEM((ROWS_HB, CHUNK_C, KPAD), jnp.float32),
        ],
    )(_weight_sc_kernel)
    weight = weight_call(sims_r)

    rows = 16
    out = pl.pallas_call(
        _fuse_tc_kernel,
        grid=(B, H // rows, HD),
        in_specs=[
            pl.BlockSpec(
                (1, 1, rows, W, K), lambda b, p, hd: (b, hd, p, 0, 0)
            ),
            pl.BlockSpec(
                (1, rows, W, KPAD), lambda b, p, hd: (b, p, 0, 0)
            ),
        ],
        out_specs=pl.BlockSpec(
            (1, 1, rows, W, K), lambda b, p, hd: (b, hd, p, 0, 0)
        ),
        out_shape=jax.ShapeDtypeStruct((B, HD, H, W, K), jnp.float32),
    )(attn, weight)
    return out


# speedup vs baseline: 3.1286x; 1.0020x over previous
import functools

import jax
import jax.numpy as jnp
from jax import lax
from jax.experimental import pallas as pl
from jax.experimental.pallas import tpu as pltpu
from jax.experimental.pallas import tpu_sc as plsc

B, HD, H, W, K = 2, 4, 224, 224, 49
SH = SW = 14
NCH = SH * SW
KSIZE, R = 7, 3
KPAD = 64

ROWS_PW = 14
ROWS_HB = 7
CHUNK_C = 28
SLAB_R = ROWS_HB + 6
SLAB_C = 40


def _weight_sc_kernel(sims_hbm, w_hbm, slab, outv):
    nc = 2
    wid = lax.axis_index("s") * nc + lax.axis_index("c")
    b = wid >> 4
    band = wid & 15
    r0 = band * ROWS_PW

    iota = lax.iota(jnp.int32, 16)
    kvs = [jnp.minimum(c * 16 + iota, K - 1) for c in range(4)]
    khv = [(kv * 9363) >> 16 for kv in kvs]
    kwv = [kv - kh * KSIZE for kv, kh in zip(kvs, khv)]

    def chunk_body(ci, _):
        hb = ci >> 3
        x0 = (ci & 7) * CHUNK_C
        rh0 = r0 + hb * ROWS_HB
        y0c = jnp.clip(rh0 - 3, 0, H - SLAB_R)
        x0a = pl.multiple_of(jnp.clip((x0 - 3) & ~7, 0, W - SLAB_C), 8)

        def dma_row(yy, _):
            y = y0c + yy
            pltpu.sync_copy(
                sims_hbm.at[b, y, pl.ds(x0a, SLAB_C), :],
                slab.at[yy],
            )
            return _

        lax.fori_loop(0, SLAB_R, dma_row, None)

        def row_body(hh, _):
            h = rh0 + hh
            sih = h >> 4
            hclip = jnp.clip(h - 3, 0, H - KSIZE)
            i0s = [hclip + khv[c] - y0c for c in range(4)]

            def col_body(wwi, _):
                w = x0 + wwi
                sjw = w >> 4
                wclip = jnp.clip(w - 3, 0, W - KSIZE)
                qrowv = jnp.full((16,), h - y0c, jnp.int32)
                qcolv = jnp.full((16,), w - x0a, jnp.int32)
                i1s = [(wclip - x0a) + kwv[c] for c in range(4)]
                accs = [jnp.zeros((16,), jnp.float32) for _ in range(4)]
                for di in (-1, 0, 1):
                    for dj in (-1, 0, 1):
                        g = (sih + di) * 14 + sjw + dj
                        lqv = jnp.full(
                            (16,), jnp.clip(g, 0, NCH - 1), jnp.int32
                        )
                        pi_vec = plsc.load_gather(slab, [qrowv, qcolv, lqv])
                        rowok = jnp.logical_and(sih + di >= 0, sih + di < SH)
                        ok = jnp.logical_and(
                            rowok,
                            jnp.logical_and(sjw + dj >= 0, sjw + dj < SW),
                        )
                        pi_d = jnp.where(ok, pi_vec, 0.0)
                        for c in range(4):
                            vals = plsc.load_gather(slab, [i0s[c], i1s[c], lqv])
                            accs[c] = accs[c] + pi_d * vals
                for c in range(4):
                    outv[hh, wwi, pl.ds(c * 16, 16)] = accs[c]
                return _

            lax.fori_loop(0, CHUNK_C, col_body, None)
            return _

        lax.fori_loop(0, ROWS_HB, row_body, None)
        pltpu.sync_copy(
            outv, w_hbm.at[b, pl.ds(rh0, ROWS_HB), pl.ds(x0, CHUNK_C), :]
        )
        return _

    lax.fori_loop(0, 16, chunk_body, None)


def _fuse_tc_kernel(attn_ref, w_ref, out_ref):
    a = attn_ref[0, 0]
    m = jnp.max(a, axis=-1, keepdims=True)
    e = jnp.exp(a - m)
    wt = w_ref[0][:, :, :K]
    o = e * wt
    s = jnp.sum(o, axis=-1, keepdims=True)
    out_ref[0, 0] = o * (1.0 / (1e-15 + s))


def kernel(attn, sims):
    sims_r = sims.reshape(B, H, W, NCH)

    mesh = plsc.VectorSubcoreMesh(core_axis_name="c", subcore_axis_name="s")
    weight_call = functools.partial(
        pl.kernel,
        mesh=mesh,
        compiler_params=pltpu.CompilerParams(
            use_tc_tiling_on_sc=False, needs_layout_passes=False
        ),
        out_type=jax.ShapeDtypeStruct((B, H, W, KPAD), jnp.float32),
        scratch_types=[
            pltpu.VMEM((SLAB_R, SLAB_C, NCH), jnp.float32),
            pltpu.VMEM((ROWS_HB, CHUNK_C, KPAD), jnp.float32),
        ],
    )(_weight_sc_kernel)
    weight = weight_call(sims_r)

    rows = 16
    out = pl.pallas_call(
        _fuse_tc_kernel,
        grid=(B, H // rows, HD),
        in_specs=[
            pl.BlockSpec(
                (1, 1, rows, W, K), lambda b, p, hd: (b, hd, p, 0, 0)
            ),
            pl.BlockSpec(
                (1, rows, W, KPAD), lambda b, p, hd: (b, p, 0, 0)
            ),
        ],
        out_specs=pl.BlockSpec(
            (1, 1, rows, W, K), lambda b, p, hd: (b, hd, p, 0, 0)
        ),
        out_shape=jax.ShapeDtypeStruct((B, HD, H, W, K), jnp.float32),
    )(attn, weight)
    return out
